# baseline (speedup 1.0000x reference)
import functools

import jax
import jax.numpy as jnp
import numpy as np
from jax import lax
from jax.experimental import pallas as pl
from jax.experimental.pallas import tpu as pltpu
from jax.experimental.pallas import tpu_sc as plsc

B = 8
DEPTH = 12
NPT = 2 ** (DEPTH + 1) - 1
N = B * NPT
H = 256
LEAF = 2 ** DEPTH
NLEAF = B * LEAF
SIZES = [2 ** (DEPTH - l) for l in range(DEPTH + 1)]
OFF = np.concatenate([np.zeros(1, dtype=np.int64),
                      np.cumsum(np.asarray(SIZES[:-1], dtype=np.int64))])

_NW = 32
_NHALF = NLEAF // 2
_BPW = _NHALF // _NW
_CH = 128
_NCH = _BPW // _CH


_NBUF = 3


def _sc_gather_body(emb_hbm, wid_hbm, out_hbm, idx_v,
                    rows_v, g0, g1, g2, w0, w1, w2):
    gs = (g0, g1, g2)
    ws = (w0, w1, w2)
    wid = lax.axis_index("s") * 2 + lax.axis_index("c")
    base = wid * _BPW
    pltpu.sync_copy(wid_hbm.at[pl.ds(wid * _NCH, _NCH)], idx_v)
    gh = [None] * _NCH
    wh = [None] * _NCH
    for k in range(_NBUF):
        gh[k] = pltpu.async_copy(emb_hbm.at[idx_v.at[k]], rows_v.at[k],
                                 gs[k])
    for k in range(_NCH):
        b = k % _NBUF
        gh[k].wait()
        wh[k] = pltpu.async_copy(rows_v.at[b],
                                 out_hbm.at[pl.ds(base + k * _CH, _CH)],
                                 ws[b])
        if k + _NBUF < _NCH:
            wh[k].wait()
            gh[k + _NBUF] = pltpu.async_copy(
                emb_hbm.at[idx_v.at[k + _NBUF]], rows_v.at[b], gs[b])
    for k in range(max(0, _NCH - _NBUF), _NCH):
        wh[k].wait()


def _sc_gather(emb, wid_leaf):
    k = pl.kernel(
        _sc_gather_body,
        out_type=jax.ShapeDtypeStruct((_NHALF, H), jnp.float32),
        mesh=plsc.VectorSubcoreMesh(core_axis_name="c", subcore_axis_name="s"),
        scratch_types=[
            pltpu.VMEM((_NCH, _CH), jnp.int32),
            pltpu.VMEM((_NBUF, _CH, H), jnp.float32),
            pltpu.SemaphoreType.DMA,
            pltpu.SemaphoreType.DMA,
            pltpu.SemaphoreType.DMA,
            pltpu.SemaphoreType.DMA,
            pltpu.SemaphoreType.DMA,
            pltpu.SemaphoreType.DMA,
        ],
    )
    return k(emb, wid_leaf.reshape(_NHALF // _CH, _CH))


def _tree_body(E_ref, mF_ref, Wiou_ref, biou_ref, Wcat_ref, bcat_ref,
               linw_ref, linb_ref, out_ref, Ah, Ac, Bh, Bc):
    linw = linw_ref[...]
    linb = linb_ref[...]
    for k in range(4):
        sl = pl.ds(k * 1024, 1024)
        e = (E_ref[sl, :] * mF_ref[sl, :]).astype(jnp.bfloat16)
        iou = jnp.dot(e, Wiou_ref[...],
                      preferred_element_type=jnp.float32) + biou_ref[...]
        i_ = iou[:, :H]
        o_ = iou[:, H:2 * H]
        u_ = iou[:, 2 * H:]
        c0 = jax.nn.sigmoid(i_) * jnp.tanh(u_)
        h0 = jax.nn.sigmoid(o_) * jnp.tanh(c0)
        Ah[sl, :] = h0
        Ac[sl, :] = c0
        out_ref[0, sl, :] = jnp.dot(h0, linw,
                                    preferred_element_type=jnp.float32) + linb
    src_h, src_c, dst_h, dst_c = Ah, Ac, Bh, Bc
    for l in range(1, DEPTH + 1):
        M = 1 << (DEPTH - l)
        hcat = src_h[0:2 * M, :].reshape(M, 2 * H).astype(jnp.bfloat16)
        ccat = src_c[0:2 * M, :].reshape(M, 2 * H)
        Z = jnp.dot(hcat, Wcat_ref[...],
                    preferred_element_type=jnp.float32) + bcat_ref[...]
        f = jax.nn.sigmoid(Z[:, :2 * H])
        cred = f[:, :H] * ccat[:, :H] + f[:, H:] * ccat[:, H:]
        i_ = Z[:, 2 * H:3 * H]
        o_ = Z[:, 3 * H:4 * H]
        u_ = Z[:, 4 * H:]
        cn = jax.nn.sigmoid(i_) * jnp.tanh(u_) + cred
        hn = jax.nn.sigmoid(o_) * jnp.tanh(cn)
        dst_h[0:M, :] = hn
        dst_c[0:M, :] = cn
        out_ref[0, pl.ds(int(OFF[l]), M), :] = (
            jnp.dot(hn, linw, preferred_element_type=jnp.float32) + linb)
        src_h, src_c, dst_h, dst_c = dst_h, dst_c, src_h, src_c


def _tree_call(E, maskf, W_iou, b_iou, Wcat, bcat, linw, linb,
               interpret=False):
    return pl.pallas_call(
        _tree_body,
        grid=(B // 2,),
        in_specs=[
            pl.BlockSpec((LEAF, H), lambda b: (b, 0)),
            pl.BlockSpec((LEAF, 1), lambda b: (b, 0)),
            pl.BlockSpec((H, 3 * H), lambda b: (0, 0)),
            pl.BlockSpec((1, 3 * H), lambda b: (0, 0)),
            pl.BlockSpec((2 * H, 5 * H), lambda b: (0, 0)),
            pl.BlockSpec((1, 5 * H), lambda b: (0, 0)),
            pl.BlockSpec((H, 8), lambda b: (0, 0)),
            pl.BlockSpec((1, 8), lambda b: (0, 0)),
        ],
        out_specs=pl.BlockSpec((1, NPT, 8), lambda b: (b, 0, 0)),
        out_shape=jax.ShapeDtypeStruct((B // 2, NPT, 8), jnp.float32),
        scratch_shapes=[
            pltpu.VMEM((LEAF, H), jnp.float32),
            pltpu.VMEM((LEAF, H), jnp.float32),
            pltpu.VMEM((LEAF // 2, H), jnp.float32),
            pltpu.VMEM((LEAF // 2, H), jnp.float32),
        ],
        interpret=interpret,
    )(E, maskf, W_iou, b_iou, Wcat, bcat, linw, linb)


def kernel(wordid, mask, h, c, emb, W_iou, U_iou, b_iou, U_f_w, U_f_b,
           lin_w, lin_b):
    wid_leaf = wordid.reshape(B, NPT)[:, :LEAF].astype(jnp.int32)
    msk_leaf = mask.reshape(B, NPT)[:, :LEAF].astype(jnp.int32)
    maskf = msk_leaf.astype(jnp.float32).reshape(NLEAF, 1)
    E1 = _sc_gather(emb, wid_leaf[:B // 2].reshape(-1))
    E2 = _sc_gather(emb, wid_leaf[B // 2:].reshape(-1))
    Wcat = jnp.concatenate([U_f_w, U_iou], axis=1).astype(jnp.bfloat16)
    bcat = jnp.concatenate([U_f_b.reshape(1, -1), b_iou], axis=1)
    linw = jnp.pad(lin_w, ((0, 0), (0, 3)))
    linb = jnp.pad(lin_b, (0, 3)).reshape(1, 8)
    Wiou_bf = W_iou.astype(jnp.bfloat16)
    out1 = _tree_call(E1, maskf[:_NHALF], Wiou_bf, b_iou, Wcat,
                      bcat, linw, linb)
    out2 = _tree_call(E2, maskf[_NHALF:], Wiou_bf, b_iou, Wcat,
                      bcat, linw, linb)
    out = jnp.concatenate([out1, out2], axis=0)
    return out.reshape(N, 8)[:, :5]


# speedup vs baseline: 1.2675x; 1.2675x over previous
import jax
import jax.numpy as jnp
import numpy as np
from jax import lax
from jax.experimental import pallas as pl
from jax.experimental.pallas import tpu as pltpu
from jax.experimental.pallas import tpu_sc as plsc

B = 8
DEPTH = 12
NPT = 2 ** (DEPTH + 1) - 1
N = B * NPT
H = 256
LEAF = 2 ** DEPTH
NLEAF = B * LEAF
SIZES = [2 ** (DEPTH - l) for l in range(DEPTH + 1)]
OFF = np.concatenate([np.zeros(1, dtype=np.int64),
                      np.cumsum(np.asarray(SIZES[:-1], dtype=np.int64))])
LSPLIT = 5
M2 = 1 << (DEPTH - LSPLIT + 1)
NLOW = int(OFF[LSPLIT])
NUP = NPT - NLOW

_NW = 32
_BPW = NLEAF // _NW
_CH = 128
_NCH = _BPW // _CH
_NBUF = 3


def _sc_gather_body(emb_hbm, wid_hbm, out_hbm, idx_v, rows_v,
                    g0, g1, g2, w0, w1, w2):
    gs = (g0, g1, g2)
    ws = (w0, w1, w2)
    wid = lax.axis_index("s") * 2 + lax.axis_index("c")
    base = wid * _BPW
    pltpu.sync_copy(wid_hbm.at[pl.ds(wid * _NCH, _NCH)], idx_v)
    gh = [None] * _NCH
    wh = [None] * _NCH
    for k in range(_NBUF):
        gh[k] = pltpu.async_copy(emb_hbm.at[idx_v.at[k]], rows_v.at[k],
                                 gs[k])
    for k in range(_NCH):
        b = k % _NBUF
        gh[k].wait()
        wh[k] = pltpu.async_copy(rows_v.at[b],
                                 out_hbm.at[pl.ds(base + k * _CH, _CH)],
                                 ws[b])
        if k + _NBUF < _NCH:
            wh[k].wait()
            gh[k + _NBUF] = pltpu.async_copy(
                emb_hbm.at[idx_v.at[k + _NBUF]], rows_v.at[b], gs[b])
    for k in range(max(0, _NCH - _NBUF), _NCH):
        wh[k].wait()


def _sc_gather(emb, wid_leaf):
    k = pl.kernel(
        _sc_gather_body,
        out_type=jax.ShapeDtypeStruct((NLEAF, H), jnp.float32),
        mesh=plsc.VectorSubcoreMesh(core_axis_name="c", subcore_axis_name="s"),
        scratch_types=[
            pltpu.VMEM((_NCH, _CH), jnp.int32),
            pltpu.VMEM((_NBUF, _CH, H), jnp.float32),
            pltpu.SemaphoreType.DMA,
            pltpu.SemaphoreType.DMA,
            pltpu.SemaphoreType.DMA,
            pltpu.SemaphoreType.DMA,
            pltpu.SemaphoreType.DMA,
            pltpu.SemaphoreType.DMA,
        ],
    )
    return k(emb, wid_leaf.reshape(NLEAF // _CH, _CH))


def _combine(hcat, ccat, Wcat_ref, bcat_ref):
    Z = jnp.dot(hcat.astype(jnp.bfloat16), Wcat_ref[...],
                preferred_element_type=jnp.float32) + bcat_ref[...]
    f = jax.nn.sigmoid(Z[:, :2 * H])
    cred = f[:, :H] * ccat[:, :H] + f[:, H:] * ccat[:, H:]
    i_ = Z[:, 2 * H:3 * H]
    o_ = Z[:, 3 * H:4 * H]
    u_ = Z[:, 4 * H:]
    cn = jax.nn.sigmoid(i_) * jnp.tanh(u_) + cred
    hn = jax.nn.sigmoid(o_) * jnp.tanh(cn)
    return hn, cn


def _tree_body(E_ref, mF_ref, Wiou_ref, biou_ref, Wcat_ref, bcat_ref,
               linw_ref, linb_ref, out_ref, out2_ref, Ah, Ac, Bh, Bc,
               H2, C2):
    b = pl.program_id(0)
    linw = linw_ref[...]
    linb = linb_ref[...]
    for k in range(4):
        sl = pl.ds(k * 1024, 1024)
        e = (E_ref[sl, :] * mF_ref[sl, :]).astype(jnp.bfloat16)
        iou = jnp.dot(e, Wiou_ref[...],
                      preferred_element_type=jnp.float32) + biou_ref[...]
        i_ = iou[:, :H]
        o_ = iou[:, H:2 * H]
        u_ = iou[:, 2 * H:]
        c0 = jax.nn.sigmoid(i_) * jnp.tanh(u_)
        h0 = jax.nn.sigmoid(o_) * jnp.tanh(c0)
        Ah[sl, :] = h0
        Ac[sl, :] = c0
        out_ref[0, sl, :] = jnp.dot(h0, linw,
                                    preferred_element_type=jnp.float32) + linb
    for i in range(2):
        h1, c1 = _combine(Ah[2048 * i:2048 * (i + 1), :].reshape(1024, 2 * H),
                          Ac[2048 * i:2048 * (i + 1), :].reshape(1024, 2 * H),
                          Wcat_ref, bcat_ref)
        Bh[1024 * i:1024 * (i + 1), :] = h1
        Bc[1024 * i:1024 * (i + 1), :] = c1
        out_ref[0, pl.ds(int(OFF[1]) + 1024 * i, 1024), :] = (
            jnp.dot(h1, linw, preferred_element_type=jnp.float32) + linb)
    h2, c2 = _combine(Bh[...].reshape(1024, 2 * H),
                      Bc[...].reshape(1024, 2 * H), Wcat_ref, bcat_ref)
    Ah[0:1024, :] = h2
    Ac[0:1024, :] = c2
    out_ref[0, pl.ds(int(OFF[2]), 1024), :] = (
        jnp.dot(h2, linw, preferred_element_type=jnp.float32) + linb)
    h3, c3 = _combine(Ah[0:1024, :].reshape(512, 2 * H),
                      Ac[0:1024, :].reshape(512, 2 * H), Wcat_ref, bcat_ref)
    Bh[0:512, :] = h3
    Bc[0:512, :] = c3
    out_ref[0, pl.ds(int(OFF[3]), 512), :] = (
        jnp.dot(h3, linw, preferred_element_type=jnp.float32) + linb)
    h4, c4 = _combine(Bh[0:512, :].reshape(M2, 2 * H),
                      Bc[0:512, :].reshape(M2, 2 * H), Wcat_ref, bcat_ref)
    H2[pl.ds(b * M2, M2), :] = h4
    C2[pl.ds(b * M2, M2), :] = c4
    out_ref[0, pl.ds(int(OFF[4]), M2), :] = (
        jnp.dot(h4, linw, preferred_element_type=jnp.float32) + linb)

    @pl.when(b == B - 1)
    def _upper():
        src_h, src_c, dst_h, dst_c = H2, C2, Ah, Ac
        for l in range(LSPLIT, DEPTH + 1):
            Mt = 1 << (DEPTH - l)
            Mg = Mt * B
            hn, cn = _combine(src_h[0:2 * Mg, :].reshape(Mg, 2 * H),
                              src_c[0:2 * Mg, :].reshape(Mg, 2 * H),
                              Wcat_ref, bcat_ref)
            dst_h[0:Mg, :] = hn
            dst_c[0:Mg, :] = cn
            lg = jnp.dot(hn, linw, preferred_element_type=jnp.float32) + linb
            off2 = int(OFF[l] - OFF[LSPLIT])
            for t in range(B):
                out2_ref[t, pl.ds(off2, Mt), :] = lg[t * Mt:(t + 1) * Mt, :]
            src_h, src_c, dst_h, dst_c = dst_h, dst_c, src_h, src_c


def _tree_call(E, maskf, W_iou, b_iou, Wcat, bcat, linw, linb,
               interpret=False):
    return pl.pallas_call(
        _tree_body,
        grid=(B,),
        in_specs=[
            pl.BlockSpec((LEAF, H), lambda b: (b, 0)),
            pl.BlockSpec((LEAF, 1), lambda b: (b, 0)),
            pl.BlockSpec((H, 3 * H), lambda b: (0, 0)),
            pl.BlockSpec((1, 3 * H), lambda b: (0, 0)),
            pl.BlockSpec((2 * H, 5 * H), lambda b: (0, 0)),
            pl.BlockSpec((1, 5 * H), lambda b: (0, 0)),
            pl.BlockSpec((H, 8), lambda b: (0, 0)),
            pl.BlockSpec((1, 8), lambda b: (0, 0)),
        ],
        out_specs=[
            pl.BlockSpec((1, NLOW, 8), lambda b: (b, 0, 0)),
            pl.BlockSpec((B, NUP, 8), lambda b: (0, 0, 0)),
        ],
        out_shape=[
            jax.ShapeDtypeStruct((B, NLOW, 8), jnp.float32),
            jax.ShapeDtypeStruct((B, NUP, 8), jnp.float32),
        ],
        scratch_shapes=[
            pltpu.VMEM((LEAF, H), jnp.float32),
            pltpu.VMEM((LEAF, H), jnp.float32),
            pltpu.VMEM((LEAF // 2, H), jnp.float32),
            pltpu.VMEM((LEAF // 2, H), jnp.float32),
            pltpu.VMEM((B * M2, H), jnp.float32),
            pltpu.VMEM((B * M2, H), jnp.float32),
        ],
        interpret=interpret,
    )(E, maskf, W_iou, b_iou, Wcat, bcat, linw, linb)


def kernel(wordid, mask, h, c, emb, W_iou, U_iou, b_iou, U_f_w, U_f_b,
           lin_w, lin_b):
    wid_leaf = wordid.reshape(B, NPT)[:, :LEAF].reshape(-1).astype(jnp.int32)
    msk_leaf = mask.reshape(B, NPT)[:, :LEAF].reshape(-1).astype(jnp.int32)
    maskf = msk_leaf.astype(jnp.float32).reshape(NLEAF, 1)
    E = _sc_gather(emb, wid_leaf)
    Wcat = jnp.concatenate([U_f_w, U_iou], axis=1).astype(jnp.bfloat16)
    bcat = jnp.concatenate([U_f_b.reshape(1, -1), b_iou], axis=1)
    linw = jnp.pad(lin_w, ((0, 0), (0, 3)))
    linb = jnp.pad(lin_b, (0, 3)).reshape(1, 8)
    out1, out2 = _tree_call(E, maskf, W_iou.astype(jnp.bfloat16), b_iou,
                            Wcat, bcat, linw, linb)
    out = jnp.concatenate([out1, out2], axis=1)
    return out.reshape(N, 8)[:, :5]
